# baseline (speedup 1.0000x reference)
import functools

import jax
import jax.numpy as jnp
from jax import lax
from jax.experimental import pallas as pl
from jax.experimental.pallas import tpu as pltpu
from jax.experimental.pallas import tpu_sc as plsc

_BBLK = 128
_NC, _NS, _L = 2, 16, 16
_NW = _NC * _NS
_RC = 32
_R = 160


def _tc_body(t0_ref, t1_ref, pe_ref, o0_ref, o1_ref):
    o0_ref[...] = t0_ref[...] + pe_ref[1, :][None, None, :]
    o1_ref[...] = t1_ref[...] + pe_ref[2, :][None, None, :]


def _tc_adds(t0, t1, pe):
    t, b, d = t0.shape
    grid = (b // _BBLK,)
    big = pl.BlockSpec((t, _BBLK, d), lambda i: (0, i, 0))
    pe_in = pl.BlockSpec((7, d), lambda i: (0, 0))
    big_shape = jax.ShapeDtypeStruct((t, b, d), jnp.float32)
    return pl.pallas_call(
        _tc_body,
        grid=grid,
        in_specs=[big, big, pe_in],
        out_specs=[big, big],
        out_shape=[big_shape, big_shape],
        compiler_params=pltpu.CompilerParams(
            dimension_semantics=("arbitrary",),
        ),
    )(t0, t1, pe)


_DNUMS = lax.GatherDimensionNumbers(
    offset_dims=(), collapsed_slice_dims=(0,), start_index_map=(0,))


def _splat(v, el):
    idx = jnp.full((_L, 1), el, jnp.int32)
    return lax.gather(v, idx, _DNUMS, (1,),
                      mode=lax.GatherScatterMode.PROMISE_IN_BOUNDS)


def _sc_body(cb, d, rows_w, rem_hbm, ridx_hbm, mtok_hbm, pe_hbm, i0_hbm,
             orv_hbm, o2_hbm,
             rem_v, idx_v, mp_v, out_v, ib0, ib1, ob0, ob1,
             isem0, isem1, osem0, osem1):
    cid = lax.axis_index("c")
    sid = lax.axis_index("s")
    wid = sid * _NC + cid
    base = wid * cb
    rbase = wid * rows_w
    nch = d // _L
    nk = rows_w // _R

    ibufs, obufs = (ib0, ib1), (ob0, ob1)
    isems, osems = (isem0, isem1), (osem0, osem1)

    def in_slice(k):
        return i0_hbm.at[pl.ds(rbase + k * _R, _R)]

    def out_slice(k):
        return o2_hbm.at[pl.ds(rbase + k * _R, _R)]

    pltpu.async_copy(in_slice(0), ibufs[0], isems[0])
    pltpu.async_copy(in_slice(1), ibufs[1], isems[1])

    pltpu.sync_copy(ridx_hbm.at[wid], idx_v)
    pltpu.sync_copy(mtok_hbm, mp_v.at[pl.ds(0, 1)])
    pltpu.sync_copy(pe_hbm, mp_v.at[pl.ds(1, 7)])

    mask_c = [mp_v[0, pl.ds(c * _L, _L)] for c in range(nch)]
    pe4_c = [mp_v[5, pl.ds(c * _L, _L)] for c in range(nch)]
    base1_c = [mask_c[c] + mp_v[6, pl.ds(c * _L, _L)] for c in range(nch)]
    base2_c = [mask_c[c] + mp_v[7, pl.ds(c * _L, _L)] for c in range(nch)]

    def rev_group(ro, g):
        gs = ro + g * _L
        s1v = jnp.where(idx_v[0, pl.ds(gs, _L)] == 0, 1.0, 0.0)
        s2v = jnp.where(idx_v[1, pl.ds(gs, _L)] == 0, 1.0, 0.0)
        s3v = jnp.where(idx_v[2, pl.ds(gs, _L)] == 0, 1.0, 0.0)
        for el in range(_L):
            e = g * _L + el
            s1 = _splat(s1v, el)
            s2 = _splat(s2v, el)
            s3 = _splat(s3v, el)
            for c in range(nch):
                sl = pl.ds(c * _L, _L)
                rem0 = rem_v[e, 0, sl]
                rem1 = rem_v[e, 1, sl]
                out_v[e, 0, sl] = rem0 + pe4_c[c]
                diff = rem1 - mask_c[c]
                out_v[e, 1, sl] = base1_c[c] + s1 * diff
                out_v[e, 2, sl] = base2_c[c] + s2 * diff
                out_v[e, 3, sl] = mask_c[c] + s3 * diff

    def rev_chunk(rk, carry):
        ro = rk * _RC
        pltpu.sync_copy(rem_hbm.at[pl.ds(base + ro, _RC)], rem_v)
        for g in range(_RC // _L):
            rev_group(ro, g)
        pltpu.sync_copy(out_v, orv_hbm.at[pl.ds(base + ro, _RC)])
        return carry

    lax.fori_loop(0, cb // _RC, rev_chunk, 0)

    pe3_c = [mp_v[4, pl.ds(c * _L, _L)] for c in range(nch)]

    def add_chunk(ib, ob):
        def rows(r, carry):
            for u in range(2):
                for c in range(nch):
                    sl = pl.ds(c * _L, _L)
                    ob[2 * r + u, sl] = ib[2 * r + u, sl] + pe3_c[c]
            return carry
        lax.fori_loop(0, _R // 2, rows, 0)

    def step(k, b, first, last):
        pltpu.make_async_copy(in_slice(k), ibufs[b], isems[b]).wait()
        if not first:
            pltpu.make_async_copy(obufs[b], out_slice(k), osems[b]).wait()
        add_chunk(ibufs[b], obufs[b])
        pltpu.async_copy(obufs[b], out_slice(k), osems[b])
        if not last:
            pltpu.async_copy(in_slice(k + 2), ibufs[b], isems[b])

    step(0, 0, True, False)
    step(1, 1, True, False)

    def super_step(s, carry):
        step(2 * s, 0, False, False)
        step(2 * s + 1, 1, False, False)
        return carry

    lax.fori_loop(1, nk // 2 - 1, super_step, 0)
    step(nk - 2, 0, False, True)
    step(nk - 1, 1, False, True)
    pltpu.make_async_copy(obufs[0], out_slice(nk - 2), osems[0]).wait()
    pltpu.make_async_copy(obufs[1], out_slice(nk - 1), osems[1]).wait()


def _sc_part(rem, ridx_t, mtok, pe, i0f):
    b = rem.shape[0]
    d = rem.shape[2]
    cb = b // _NW
    rows_w = i0f.shape[0] // _NW
    mesh = plsc.VectorSubcoreMesh(core_axis_name="c", subcore_axis_name="s",
                                  num_cores=_NC, num_subcores=_NS)
    f = pl.kernel(
        functools.partial(_sc_body, cb, d, rows_w),
        out_type=[
            jax.ShapeDtypeStruct((b, 4, d), jnp.float32),
            jax.ShapeDtypeStruct(i0f.shape, jnp.float32),
        ],
        mesh=mesh,
        scratch_types=[
            pltpu.VMEM((_RC, 2, d), jnp.float32),
            pltpu.VMEM((3, cb), jnp.int32),
            pltpu.VMEM((8, d), jnp.float32),
            pltpu.VMEM((_RC, 4, d), jnp.float32),
            pltpu.VMEM((_R, d), jnp.float32),
            pltpu.VMEM((_R, d), jnp.float32),
            pltpu.VMEM((_R, d), jnp.float32),
            pltpu.VMEM((_R, d), jnp.float32),
            pltpu.SemaphoreType.DMA,
            pltpu.SemaphoreType.DMA,
            pltpu.SemaphoreType.DMA,
            pltpu.SemaphoreType.DMA,
        ],
    )
    return f(rem, ridx_t, mtok, pe, i0f)


def kernel(temporal_t0, temporal_t1, img_i0, others_remain_data, mask_token,
           revert_idx, pos_emb):
    b, t, d = img_i0.shape
    ridx_t = revert_idx.T.reshape(3, _NW, b // _NW).transpose(1, 0, 2)
    i0f = img_i0.transpose(1, 0, 2).reshape(t * b, d)
    orv, o2f = _sc_part(others_remain_data, ridx_t, mask_token, pos_emb, i0f)
    o2 = o2f.reshape(t, b, d).transpose(1, 0, 2)
    o0, o1 = _tc_adds(temporal_t0.transpose(1, 0, 2),
                      temporal_t1.transpose(1, 0, 2), pos_emb)
    return (o0.transpose(1, 0, 2), o1.transpose(1, 0, 2), o2, orv)


# speedup vs baseline: 1.0391x; 1.0391x over previous
import functools

import jax
import jax.numpy as jnp
from jax import lax
from jax.experimental import pallas as pl
from jax.experimental.pallas import tpu as pltpu
from jax.experimental.pallas import tpu_sc as plsc

_BBLK = 128
_NC, _NS, _L = 2, 16, 16
_NW = _NC * _NS
_RC = 32


def _tc_body(t0_ref, t1_ref, i0_ref, pe_ref, o0_ref, o1_ref, o2_ref):
    o0_ref[...] = t0_ref[...] + pe_ref[1, :][None, None, :]
    o1_ref[...] = t1_ref[...] + pe_ref[2, :][None, None, :]
    o2_ref[...] = i0_ref[...] + pe_ref[3, :][None, None, :]


def _tc_adds(t0, t1, i0, pe):
    t, b, d = t0.shape
    grid = (b // _BBLK,)
    big = pl.BlockSpec((t, _BBLK, d), lambda i: (0, i, 0))
    pe_in = pl.BlockSpec((7, d), lambda i: (0, 0))
    big_shape = jax.ShapeDtypeStruct((t, b, d), jnp.float32)
    return pl.pallas_call(
        _tc_body,
        grid=grid,
        in_specs=[big, big, big, pe_in],
        out_specs=[big, big, big],
        out_shape=[big_shape, big_shape, big_shape],
        compiler_params=pltpu.CompilerParams(
            dimension_semantics=("arbitrary",),
        ),
    )(t0, t1, i0, pe)


_DNUMS = lax.GatherDimensionNumbers(
    offset_dims=(), collapsed_slice_dims=(0,), start_index_map=(0,))


def _splat(v, el):
    idx = jnp.full((_L, 1), el, jnp.int32)
    return lax.gather(v, idx, _DNUMS, (1,),
                      mode=lax.GatherScatterMode.PROMISE_IN_BOUNDS)


def _sc_body(cb, d, rem_hbm, ridx_hbm, mtok_hbm, pe_hbm, orv_hbm,
             rem_v, idx_v, mp_v, out_v):
    cid = lax.axis_index("c")
    sid = lax.axis_index("s")
    wid = sid * _NC + cid
    base = wid * cb
    nch = d // _L

    pltpu.sync_copy(ridx_hbm.at[wid], idx_v)
    pltpu.sync_copy(mtok_hbm, mp_v.at[pl.ds(0, 1)])
    pltpu.sync_copy(pe_hbm, mp_v.at[pl.ds(1, 7)])

    mask_c = [mp_v[0, pl.ds(c * _L, _L)] for c in range(nch)]
    pe4_c = [mp_v[5, pl.ds(c * _L, _L)] for c in range(nch)]
    base1_c = [mask_c[c] + mp_v[6, pl.ds(c * _L, _L)] for c in range(nch)]
    base2_c = [mask_c[c] + mp_v[7, pl.ds(c * _L, _L)] for c in range(nch)]

    def rev_group(ro, g):
        gs = ro + g * _L
        s1v = jnp.where(idx_v[0, pl.ds(gs, _L)] == 0, 1.0, 0.0)
        s2v = jnp.where(idx_v[1, pl.ds(gs, _L)] == 0, 1.0, 0.0)
        s3v = jnp.where(idx_v[2, pl.ds(gs, _L)] == 0, 1.0, 0.0)
        for el in range(_L):
            e = g * _L + el
            s1 = _splat(s1v, el)
            s2 = _splat(s2v, el)
            s3 = _splat(s3v, el)
            for c in range(nch):
                sl = pl.ds(c * _L, _L)
                rem0 = rem_v[e, 0, sl]
                rem1 = rem_v[e, 1, sl]
                out_v[e, 0, sl] = rem0 + pe4_c[c]
                diff = rem1 - mask_c[c]
                out_v[e, 1, sl] = base1_c[c] + s1 * diff
                out_v[e, 2, sl] = base2_c[c] + s2 * diff
                out_v[e, 3, sl] = mask_c[c] + s3 * diff

    def rev_chunk(rk, carry):
        ro = rk * _RC
        pltpu.sync_copy(rem_hbm.at[pl.ds(base + ro, _RC)], rem_v)
        for g in range(_RC // _L):
            rev_group(ro, g)
        pltpu.sync_copy(out_v, orv_hbm.at[pl.ds(base + ro, _RC)])
        return carry

    lax.fori_loop(0, cb // _RC, rev_chunk, 0)


def _sc_revert(rem, ridx_t, mtok, pe):
    b = rem.shape[0]
    d = rem.shape[2]
    cb = b // _NW
    mesh = plsc.VectorSubcoreMesh(core_axis_name="c", subcore_axis_name="s",
                                  num_cores=_NC, num_subcores=_NS)
    f = pl.kernel(
        functools.partial(_sc_body, cb, d),
        out_type=jax.ShapeDtypeStruct((b, 4, d), jnp.float32),
        mesh=mesh,
        scratch_types=[
            pltpu.VMEM((_RC, 2, d), jnp.float32),
            pltpu.VMEM((3, cb), jnp.int32),
            pltpu.VMEM((8, d), jnp.float32),
            pltpu.VMEM((_RC, 4, d), jnp.float32),
        ],
    )
    return f(rem, ridx_t, mtok, pe)


def kernel(temporal_t0, temporal_t1, img_i0, others_remain_data, mask_token,
           revert_idx, pos_emb):
    b = revert_idx.shape[0]
    ridx_t = revert_idx.T.reshape(3, _NW, b // _NW).transpose(1, 0, 2)
    orv = _sc_revert(others_remain_data, ridx_t, mask_token, pos_emb)
    o0, o1, o2 = _tc_adds(temporal_t0.transpose(1, 0, 2),
                          temporal_t1.transpose(1, 0, 2),
                          img_i0.transpose(1, 0, 2), pos_emb)
    return (o0.transpose(1, 0, 2), o1.transpose(1, 0, 2),
            o2.transpose(1, 0, 2), orv)
